# baseline (speedup 1.0000x reference)
import functools

import jax
import jax.numpy as jnp
from jax import lax
from jax.experimental import pallas as pl
from jax.experimental.pallas import tpu as pltpu
from jax.experimental.pallas import tpu_sc as plsc

_N = 10000
_D = 128
_E = 320000
_NC = 2
_NS = 16
_NW = _NC * _NS
_CH = 128
_NCHUNK = 80
_NPASS = 2
_CPP = _NCHUNK // _NPASS
_EPP = _CPP * _CH
_EPW = _NCHUNK * _CH
_EPAD = _NW * _EPW
_RSPAN = 624
_TAIL = _N - _NS * _RSPAN
_ZSIZES = (128, 128, 128, 128, 112)
_NLANE = _D // 16


@functools.partial(
    pl.kernel,
    out_type=jax.ShapeDtypeStruct((_NC, _N, _D), jnp.float32),
    mesh=plsc.VectorSubcoreMesh(core_axis_name="c", subcore_axis_name="s"),
    scratch_types=[
        pltpu.VMEM((_EPP,), jnp.int32),
        pltpu.VMEM((_CPP, _CH), jnp.int32),
        pltpu.VMEM((_EPP,), jnp.float32),
        pltpu.VMEM((_CH, _D), jnp.float32),
        pltpu.VMEM((_CH, _D), jnp.float32),
        pltpu.VMEM_SHARED((_N, _D), jnp.float32),
        pltpu.SemaphoreType.DMA,
        pltpu.SemaphoreType.DMA,
    ],
)
def _sc_aggregate(x_hbm, src_hbm, dst2_hbm, w_hbm, out_hbm,
                  src_v, dst_v, w_v, rows_a, rows_b, acc, sem_a, sem_b):
    c = lax.axis_index("c")
    s = lax.axis_index("s")
    wid = c * _NS + s

    zero16 = jnp.zeros((16,), jnp.float32)

    def _zero_row(i, carry):
        for k in range(_NLANE):
            rows_a[i, pl.ds(k * 16, 16)] = zero16
        return carry

    lax.fori_loop(0, _CH, _zero_row, 0)
    zoff = 0
    for zsz in _ZSIZES:
        pltpu.sync_copy(rows_a.at[pl.ds(0, zsz)],
                        acc.at[pl.ds(s * _RSPAN + zoff, zsz)])
        zoff += zsz

    @pl.when(s == _NS - 1)
    def _zero_tail():
        pltpu.sync_copy(rows_a.at[pl.ds(0, _TAIL)],
                        acc.at[pl.ds(_NS * _RSPAN, _TAIL)])

    plsc.subcore_barrier()

    def _start_gather(i, rows_ref, sem):
        idx = src_v.at[pl.ds(i * _CH, _CH)]
        return pltpu.async_copy(x_hbm.at[idx], rows_ref, sem)

    def _wait_gather(rows_ref, sem):
        pltpu.make_async_copy(x_hbm.at[src_v.at[pl.ds(0, _CH)]],
                              rows_ref, sem).wait()

    def _scale(rows_ref, i):
        def _one_group(g, carry):
            wv = w_v[pl.ds(i * _CH + g * 16, 16)]
            for j in range(16):
                e = g * 16 + j
                wb = jnp.full((16,), wv[j], jnp.float32)
                for k in range(_NLANE):
                    rows_ref[e, pl.ds(k * 16, 16)] = (
                        rows_ref[e, pl.ds(k * 16, 16)] * wb)
            return carry
        lax.fori_loop(0, _CH // 16, _one_group, 0)

    def _scatter(rows_ref, i):
        pltpu.sync_copy(rows_ref, acc.at[dst_v.at[i]], add=True)

    for h in range(_NPASS):
        pltpu.sync_copy(src_hbm.at[pl.ds(wid * _EPW + h * _EPP, _EPP)], src_v)
        pltpu.sync_copy(dst2_hbm.at[pl.ds(wid * _NCHUNK + h * _CPP, _CPP)],
                        dst_v)
        pltpu.sync_copy(w_hbm.at[pl.ds(wid * _EPW + h * _EPP, _EPP)], w_v)

        def _chunk(i, carry):
            _start_gather(i, rows_a, sem_a).wait()
            _scale(rows_a, i)
            _scatter(rows_a, i)
            return carry

        lax.fori_loop(0, _CPP, _chunk, 0)

    plsc.subcore_barrier()
    pltpu.sync_copy(acc.at[pl.ds(s * _RSPAN, _RSPAN)],
                    out_hbm.at[c, pl.ds(s * _RSPAN, _RSPAN)])

    @pl.when(s == _NS - 1)
    def _publish_tail():
        pltpu.sync_copy(acc.at[pl.ds(_NS * _RSPAN, _TAIL)],
                        out_hbm.at[c, pl.ds(_NS * _RSPAN, _TAIL)])


_BM = 1000


def _tc_body(p_ref, w_ref, o_ref):
    agg = p_ref[0] + p_ref[1]
    o_ref[...] = jnp.maximum(
        jnp.dot(agg, w_ref[...], preferred_element_type=jnp.float32), 0.0)


def _tc_matmul_relu(partials, W):
    return pl.pallas_call(
        _tc_body,
        grid=(_N // _BM,),
        in_specs=[
            pl.BlockSpec((_NC, _BM, _D), lambda i: (0, i, 0)),
            pl.BlockSpec((_D, _D), lambda i: (0, 0)),
        ],
        out_specs=pl.BlockSpec((_BM, _D), lambda i: (i, 0)),
        out_shape=jax.ShapeDtypeStruct((_N, _D), jnp.float32),
    )(partials, W)


@jax.jit
def kernel(x, edge_index, edge_weight, W):
    src = edge_index[0].astype(jnp.int32)
    dst = edge_index[1].astype(jnp.int32)
    w = edge_weight.astype(jnp.float32)
    npad = _EPAD - _E
    src = jnp.concatenate([src, jnp.zeros((npad,), jnp.int32)])
    dst = jnp.concatenate([dst, jnp.zeros((npad,), jnp.int32)])
    w = jnp.concatenate([w, jnp.zeros((npad,), jnp.float32)])
    dst2 = dst.reshape(_EPAD // _CH, _CH)
    partials = _sc_aggregate(x, src, dst2, w)
    return _tc_matmul_relu(partials, W)


# speedup vs baseline: 1.1437x; 1.1437x over previous
import functools

import jax
import jax.numpy as jnp
from jax import lax
from jax.experimental import pallas as pl
from jax.experimental.pallas import tpu as pltpu
from jax.experimental.pallas import tpu_sc as plsc

_N = 10000
_D = 128
_E = 320000
_NC = 2
_NS = 16
_NW = _NC * _NS
_CH = 128
_NCHUNK = 80
_EPW = _NCHUNK * _CH
_EPAD = _NW * _EPW
_RSPAN = 624
_TAIL = _N - _NS * _RSPAN
_ZSIZES = (128, 128, 128, 128, 112)
_NLANE = _D // 16


@functools.partial(
    pl.kernel,
    out_type=jax.ShapeDtypeStruct((_NC, _N, _D), jnp.float32),
    mesh=plsc.VectorSubcoreMesh(core_axis_name="c", subcore_axis_name="s"),
    scratch_types=[
        pltpu.VMEM((_CH,), jnp.int32),
        pltpu.VMEM((_CH,), jnp.int32),
        pltpu.VMEM((_CH,), jnp.int32),
        pltpu.VMEM((_CH,), jnp.int32),
        pltpu.VMEM((_CH,), jnp.float32),
        pltpu.VMEM((_CH,), jnp.float32),
        pltpu.VMEM((_CH, _D), jnp.float32),
        pltpu.VMEM((_CH, _D), jnp.float32),
        pltpu.VMEM_SHARED((_N, _D), jnp.float32),
        pltpu.SemaphoreType.DMA,
        pltpu.SemaphoreType.DMA,
        pltpu.SemaphoreType.DMA,
        pltpu.SemaphoreType.DMA,
    ],
)
def _sc_aggregate(x_hbm, src_hbm, dst_hbm, w_hbm, out_hbm,
                  src_a, src_b, dst_a, dst_b, w_a, w_b, rows_a, rows_b,
                  acc, sem_ia, sem_ib, sem_ga, sem_gb):
    c = lax.axis_index("c")
    s = lax.axis_index("s")
    wid = c * _NS + s
    base = wid * _EPW

    zero16 = jnp.zeros((16,), jnp.float32)

    def _zero_row(i, carry):
        for k in range(_NLANE):
            rows_a[i, pl.ds(k * 16, 16)] = zero16
        return carry

    lax.fori_loop(0, _CH, _zero_row, 0)
    zoff = 0
    for zsz in _ZSIZES:
        pltpu.sync_copy(rows_a.at[pl.ds(0, zsz)],
                        acc.at[pl.ds(s * _RSPAN + zoff, zsz)])
        zoff += zsz

    @pl.when(s == _NS - 1)
    def _zero_tail():
        pltpu.sync_copy(rows_a.at[pl.ds(0, _TAIL)],
                        acc.at[pl.ds(_NS * _RSPAN, _TAIL)])

    plsc.subcore_barrier()

    def _issue_idx(i, srcb, dstb, wb, sem):
        off = base + i * _CH
        pltpu.async_copy(src_hbm.at[pl.ds(off, _CH)], srcb, sem)
        pltpu.async_copy(dst_hbm.at[pl.ds(off, _CH)], dstb, sem)
        pltpu.async_copy(w_hbm.at[pl.ds(off, _CH)], wb, sem)

    def _wait_idx(srcb, dstb, wb, sem):
        pltpu.make_async_copy(src_hbm.at[pl.ds(0, _CH)], srcb, sem).wait()
        pltpu.make_async_copy(dst_hbm.at[pl.ds(0, _CH)], dstb, sem).wait()
        pltpu.make_async_copy(w_hbm.at[pl.ds(0, _CH)], wb, sem).wait()

    def _issue_gather(srcb, rowsb, sem):
        pltpu.async_copy(x_hbm.at[srcb], rowsb, sem)

    def _wait_gather(srcb, rowsb, sem):
        pltpu.make_async_copy(x_hbm.at[srcb], rowsb, sem).wait()

    def _scale(rows_ref, w_ref):
        def _one_group(g, carry):
            wv = w_ref[pl.ds(g * 16, 16)]
            for j in range(16):
                e = g * 16 + j
                wb = jnp.full((16,), wv[j], jnp.float32)
                for k in range(_NLANE):
                    rows_ref[e, pl.ds(k * 16, 16)] = (
                        rows_ref[e, pl.ds(k * 16, 16)] * wb)
            return carry
        lax.fori_loop(0, _CH // 16, _one_group, 0)

    def _scatter(rows_ref, dst_ref):
        pltpu.sync_copy(rows_ref, acc.at[dst_ref], add=True)

    last = _NCHUNK - 1
    pltpu.sync_copy(src_hbm.at[pl.ds(base, _CH)], src_a)
    pltpu.sync_copy(dst_hbm.at[pl.ds(base, _CH)], dst_a)
    pltpu.sync_copy(w_hbm.at[pl.ds(base, _CH)], w_a)
    _issue_gather(src_a, rows_a, sem_ga)
    _issue_idx(1, src_b, dst_b, w_b, sem_ib)

    def _pair(p, carry):
        i0 = 2 * p
        _wait_gather(src_a, rows_a, sem_ga)
        _wait_idx(src_b, dst_b, w_b, sem_ib)
        _issue_gather(src_b, rows_b, sem_gb)
        _scale(rows_a, w_a)
        _scatter(rows_a, dst_a)
        _issue_idx(jnp.minimum(i0 + 2, last), src_a, dst_a, w_a, sem_ia)
        _wait_gather(src_b, rows_b, sem_gb)
        _wait_idx(src_a, dst_a, w_a, sem_ia)
        _issue_gather(src_a, rows_a, sem_ga)
        _scale(rows_b, w_b)
        _scatter(rows_b, dst_b)
        _issue_idx(jnp.minimum(i0 + 3, last), src_b, dst_b, w_b, sem_ib)
        return carry

    lax.fori_loop(0, _NCHUNK // 2, _pair, 0)
    _wait_gather(src_a, rows_a, sem_ga)
    _wait_idx(src_b, dst_b, w_b, sem_ib)

    plsc.subcore_barrier()
    pltpu.sync_copy(acc.at[pl.ds(s * _RSPAN, _RSPAN)],
                    out_hbm.at[c, pl.ds(s * _RSPAN, _RSPAN)])

    @pl.when(s == _NS - 1)
    def _publish_tail():
        pltpu.sync_copy(acc.at[pl.ds(_NS * _RSPAN, _TAIL)],
                        out_hbm.at[c, pl.ds(_NS * _RSPAN, _TAIL)])


_BM = 1000


def _tc_body(p_ref, w_ref, o_ref):
    agg = p_ref[0] + p_ref[1]
    o_ref[...] = jnp.maximum(
        jnp.dot(agg, w_ref[...], preferred_element_type=jnp.float32), 0.0)


def _tc_matmul_relu(partials, W):
    return pl.pallas_call(
        _tc_body,
        grid=(_N // _BM,),
        in_specs=[
            pl.BlockSpec((_NC, _BM, _D), lambda i: (0, i, 0)),
            pl.BlockSpec((_D, _D), lambda i: (0, 0)),
        ],
        out_specs=pl.BlockSpec((_BM, _D), lambda i: (i, 0)),
        out_shape=jax.ShapeDtypeStruct((_N, _D), jnp.float32),
    )(partials, W)


@jax.jit
def kernel(x, edge_index, edge_weight, W):
    src = edge_index[0].astype(jnp.int32)
    dst = edge_index[1].astype(jnp.int32)
    w = edge_weight.astype(jnp.float32)
    npad = _EPAD - _E
    src = jnp.concatenate([src, jnp.zeros((npad,), jnp.int32)])
    dst = jnp.concatenate([dst, jnp.zeros((npad,), jnp.int32)])
    w = jnp.concatenate([w, jnp.zeros((npad,), jnp.float32)])
    partials = _sc_aggregate(x, src, dst, w)
    return _tc_matmul_relu(partials, W)
